# baseline (speedup 1.0000x reference)
import functools

import jax
import jax.numpy as jnp
from jax import lax
from jax.experimental import pallas as pl
from jax.experimental.pallas import tpu as pltpu
from jax.experimental.pallas import tpu_sc as plsc

B = 16384
D = 32
NC = 2
NS = 16
NW = NC * NS
BPW = B // NW
CHUNK = 128
NCHUNK = BPW // CHUNK


def _sc_gather_gmf(user2d, item2d, gmf_u_w, gmf_i_w, mlp_u_w, mlp_i_w):
    mesh = plsc.VectorSubcoreMesh(core_axis_name="c", subcore_axis_name="s")

    @functools.partial(
        pl.kernel,
        mesh=mesh,
        compiler_params=pltpu.CompilerParams(use_tc_tiling_on_sc=False),
        out_type=(
            jax.ShapeDtypeStruct((B, D), jnp.float32),
            jax.ShapeDtypeStruct((B, D), jnp.float32),
            jax.ShapeDtypeStruct((B, D), jnp.float32),
        ),
        scratch_types=[
            pltpu.VMEM((NCHUNK, CHUNK), jnp.int32),
            pltpu.VMEM((NCHUNK, CHUNK), jnp.int32),
            pltpu.VMEM((BPW, D), jnp.float32),
            pltpu.VMEM((BPW, D), jnp.float32),
            pltpu.VMEM((BPW, D), jnp.float32),
            pltpu.VMEM((BPW, D), jnp.float32),
            pltpu.SemaphoreType.DMA,
        ],
    )
    def k(user_ref, item_ref, guw, giw, muw, miw,
          gmf_out, mu_out, mi_out,
          idx_u, idx_i, gu_v, gi_v, mu_v, mi_v, sem):
        wid = lax.axis_index("s") * NC + lax.axis_index("c")
        base = wid * BPW
        row0 = wid * NCHUNK

        pltpu.sync_copy(user_ref.at[pl.ds(row0, NCHUNK)], idx_u)
        pltpu.sync_copy(item_ref.at[pl.ds(row0, NCHUNK)], idx_i)

        copies = []
        for c in range(NCHUNK):
            sl = pl.ds(c * CHUNK, CHUNK)
            copies.append(pltpu.async_copy(guw.at[idx_u.at[c]], gu_v.at[sl], sem))
            copies.append(pltpu.async_copy(giw.at[idx_i.at[c]], gi_v.at[sl], sem))
            copies.append(pltpu.async_copy(muw.at[idx_u.at[c]], mu_v.at[sl], sem))
            copies.append(pltpu.async_copy(miw.at[idx_i.at[c]], mi_v.at[sl], sem))
        for cp in copies:
            cp.wait()

        def body(j, carry):
            for c0 in range(0, D, 16):
                s = pl.ds(c0, 16)
                gu_v[j, s] = gu_v[j, s] * gi_v[j, s]
            return carry

        lax.fori_loop(0, BPW, body, 0, unroll=4)

        osl = pl.ds(base, BPW)
        pltpu.sync_copy(gu_v, gmf_out.at[osl])
        pltpu.sync_copy(mu_v, mu_out.at[osl])
        pltpu.sync_copy(mi_v, mi_out.at[osl])

    return k(user2d, item2d, gmf_u_w, gmf_i_w, mlp_u_w, mlp_i_w)


def _tc_mlp_body(gmf, mu, mi, w1t, w2t, b1, b2, wo, bo, out):
    h1 = jnp.dot(mu[...], w1t[...][:D], preferred_element_type=jnp.float32)
    h1 = h1 + jnp.dot(mi[...], w1t[...][D:], preferred_element_type=jnp.float32)
    h1 = jnp.maximum(h1 + b1[...], 0.0)
    h2 = jnp.maximum(
        jnp.dot(h1, w2t[...], preferred_element_type=jnp.float32) + b2[...], 0.0)
    o = jnp.dot(gmf[...], wo[...][:D], preferred_element_type=jnp.float32)
    o = o + jnp.dot(h2, wo[...][D:], preferred_element_type=jnp.float32)
    out[...] = o + bo[0, 0]


def _tc_mlp(gmf, mu, mi, w1t, w2t, b1, b2, wo, bo):
    BLK = 2048
    grid = B // BLK
    full = lambda shape: pl.BlockSpec(shape, lambda i: (0, 0))
    return pl.pallas_call(
        _tc_mlp_body,
        grid=(grid,),
        in_specs=[
            pl.BlockSpec((BLK, D), lambda i: (i, 0)),
            pl.BlockSpec((BLK, D), lambda i: (i, 0)),
            pl.BlockSpec((BLK, D), lambda i: (i, 0)),
            full((2 * D, 64)),
            full((64, D)),
            full((1, 64)),
            full((1, D)),
            full((2 * D, 1)),
            full((1, 1)),
        ],
        out_specs=pl.BlockSpec((BLK, 1), lambda i: (i, 0)),
        out_shape=jax.ShapeDtypeStruct((B, 1), jnp.float32),
    )(gmf, mu, mi, w1t, w2t, b1, b2, wo, bo)


def kernel(user, item, gmf_user_w, gmf_item_w, mlp_user_w, mlp_item_w,
           W1, b1, W2, b2, Wo, bo):
    user2d = user.astype(jnp.int32).reshape(NW * NCHUNK, CHUNK)
    item2d = item.astype(jnp.int32).reshape(NW * NCHUNK, CHUNK)

    gmf, mu, mi = _sc_gather_gmf(
        user2d, item2d, gmf_user_w, gmf_item_w, mlp_user_w, mlp_item_w)

    w1t = W1.T
    w2t = W2.T
    wo = Wo.T
    out = _tc_mlp(gmf, mu, mi, w1t, w2t,
                  b1.reshape(1, -1), b2.reshape(1, -1), wo, bo.reshape(1, 1))
    return out[:, 0]


# speedup vs baseline: 1.7063x; 1.7063x over previous
import functools

import jax
import jax.numpy as jnp
from jax import lax
from jax.experimental import pallas as pl
from jax.experimental.pallas import tpu as pltpu
from jax.experimental.pallas import tpu_sc as plsc

B = 16384
NU = 1000000
D = 32
NC = 2
NS = 16
NW = NC * NS
BPW = B // NW
CHUNK = 128
NCHUNK = BPW // CHUNK
RK = 8192


def _tc_repack_body(guT, muT, giT, miT, out):
    eye = jnp.eye(D, dtype=jnp.float32)
    dims = (((0,), (0,)), ((), ()))
    out[:, 0:D] = jnp.transpose(guT[...])
    out[:, D:2 * D] = lax.dot_general(muT[...], eye, dims,
                                      preferred_element_type=jnp.float32)
    out[:, 2 * D:3 * D] = jnp.transpose(giT[...])
    out[:, 3 * D:4 * D] = lax.dot_general(miT[...], eye, dims,
                                          preferred_element_type=jnp.float32)


def _tc_repack(gu, gi, mu, mi):
    grid = (NU + RK - 1) // RK
    tspec = pl.BlockSpec((D, RK), lambda i: (0, i))
    return pl.pallas_call(
        _tc_repack_body,
        grid=(grid,),
        in_specs=[tspec, tspec, tspec, tspec],
        out_specs=pl.BlockSpec((RK, 4 * D), lambda i: (i, 0)),
        out_shape=jax.ShapeDtypeStruct((NU, 4 * D), jnp.float32),
    )(gu.T, mu.T, gi.T, mi.T)


def _sc_gather(user, item, mega):
    mesh = plsc.VectorSubcoreMesh(core_axis_name="c", subcore_axis_name="s")

    @functools.partial(
        pl.kernel,
        mesh=mesh,
        compiler_params=pltpu.CompilerParams(use_tc_tiling_on_sc=True),
        out_type=(
            jax.ShapeDtypeStruct((B, 4 * D), jnp.float32),
            jax.ShapeDtypeStruct((B, 4 * D), jnp.float32),
        ),
        scratch_types=[
            pltpu.VMEM((NCHUNK, CHUNK), jnp.int32),
            pltpu.VMEM((NCHUNK, CHUNK), jnp.int32),
            pltpu.VMEM((BPW, 4 * D), jnp.float32),
            pltpu.SemaphoreType.DMA,
        ],
    )
    def k(user_ref, item_ref, mega_ref, fu_out, fi_out, idx_u, idx_i, rows_v, sem):
        wid = lax.axis_index("s") * NC + lax.axis_index("c")
        base = wid * BPW
        row0 = wid * NCHUNK

        pltpu.sync_copy(user_ref.at[pl.ds(row0, NCHUNK)], idx_u)
        pltpu.sync_copy(item_ref.at[pl.ds(row0, NCHUNK)], idx_i)

        for idx, out in ((idx_u, fu_out), (idx_i, fi_out)):
            copies = []
            for c in range(NCHUNK):
                copies.append(pltpu.async_copy(
                    mega_ref.at[idx.at[c]],
                    rows_v.at[pl.ds(c * CHUNK, CHUNK)],
                    sem))
            for cp in copies:
                cp.wait()
            pltpu.sync_copy(rows_v, out.at[pl.ds(base, BPW)])

    return k(user, item, mega)


def _tc_mlp_body(fu, fi, w1t, w2t, b1, b2, wo, bo, out):
    gmf = fu[:, 0:D] * fi[:, 2 * D:3 * D]
    h1 = jnp.dot(fu[:, D:2 * D], w1t[...][:D],
                 preferred_element_type=jnp.float32)
    h1 = h1 + jnp.dot(fi[:, 3 * D:4 * D], w1t[...][D:],
                      preferred_element_type=jnp.float32)
    h1 = jnp.maximum(h1 + b1[...], 0.0)
    h2 = jnp.maximum(
        jnp.dot(h1, w2t[...], preferred_element_type=jnp.float32) + b2[...], 0.0)
    o = jnp.dot(gmf, wo[...][:D], preferred_element_type=jnp.float32)
    o = o + jnp.dot(h2, wo[...][D:], preferred_element_type=jnp.float32)
    out[...] = o + bo[0, 0]


def _tc_mlp(fu, fi, w1t, w2t, b1, b2, wo, bo):
    BLK = 2048
    grid = B // BLK
    full = lambda shape: pl.BlockSpec(shape, lambda i: (0, 0))
    return pl.pallas_call(
        _tc_mlp_body,
        grid=(grid,),
        in_specs=[
            pl.BlockSpec((BLK, 4 * D), lambda i: (i, 0)),
            pl.BlockSpec((BLK, 4 * D), lambda i: (i, 0)),
            full((2 * D, 64)),
            full((64, D)),
            full((1, 64)),
            full((1, D)),
            full((2 * D, 1)),
            full((1, 1)),
        ],
        out_specs=pl.BlockSpec((BLK, 1), lambda i: (i, 0)),
        out_shape=jax.ShapeDtypeStruct((B, 1), jnp.float32),
    )(fu, fi, w1t, w2t, b1, b2, wo, bo)


def kernel(user, item, gmf_user_w, gmf_item_w, mlp_user_w, mlp_item_w,
           W1, b1, W2, b2, Wo, bo):
    mega = _tc_repack(gmf_user_w, gmf_item_w, mlp_user_w, mlp_item_w)

    user2d = user.astype(jnp.int32).reshape(NW * NCHUNK, CHUNK)
    item2d = item.astype(jnp.int32).reshape(NW * NCHUNK, CHUNK)
    fu, fi = _sc_gather(user2d, item2d, mega)

    w1t = W1.T
    w2t = W2.T
    wo = Wo.T
    out = _tc_mlp(fu, fi, w1t, w2t,
                  b1.reshape(1, -1), b2.reshape(1, -1), wo, bo.reshape(1, 1))
    return out[:, 0]


# speedup vs baseline: 3.1783x; 1.8627x over previous
import functools

import jax
import jax.numpy as jnp
from jax import lax
from jax.experimental import pallas as pl
from jax.experimental.pallas import tpu as pltpu
from jax.experimental.pallas import tpu_sc as plsc

B = 16384
NU = 1000000
D = 32
NC = 2
NS = 16
NW = NC * NS
BPW = B // NW
CHUNK = 128
NCHUNK = BPW // CHUNK
RK = 8192


def _tc_repack_body(guT, muT, giT, miT, out):
    for t, r in enumerate((guT, muT, giT, miT)):
        at = jnp.transpose(r[...].astype(jnp.bfloat16))
        out[:, t * D:(t + 1) * D] = at.astype(jnp.float32)


def _tc_repack(gu, gi, mu, mi):
    grid = (NU + RK - 1) // RK
    tspec = pl.BlockSpec((D, RK), lambda i: (0, i))
    return pl.pallas_call(
        _tc_repack_body,
        grid=(grid,),
        in_specs=[tspec, tspec, tspec, tspec],
        out_specs=pl.BlockSpec((RK, 4 * D), lambda i: (i, 0)),
        out_shape=jax.ShapeDtypeStruct((NU, 4 * D), jnp.float32),
    )(gu.T, mu.T, gi.T, mi.T)


def _sc_gather(user, item, mega):
    mesh = plsc.VectorSubcoreMesh(core_axis_name="c", subcore_axis_name="s")

    @functools.partial(
        pl.kernel,
        mesh=mesh,
        compiler_params=pltpu.CompilerParams(use_tc_tiling_on_sc=True),
        out_type=(
            jax.ShapeDtypeStruct((B, 4 * D), jnp.float32),
            jax.ShapeDtypeStruct((B, 4 * D), jnp.float32),
        ),
        scratch_types=[
            pltpu.VMEM((NCHUNK, CHUNK), jnp.int32),
            pltpu.VMEM((NCHUNK, CHUNK), jnp.int32),
            pltpu.VMEM((BPW, 4 * D), jnp.float32),
            pltpu.SemaphoreType.DMA,
        ],
    )
    def k(user_ref, item_ref, mega_ref, fu_out, fi_out, idx_u, idx_i, rows_v, sem):
        wid = lax.axis_index("s") * NC + lax.axis_index("c")
        base = wid * BPW
        row0 = wid * NCHUNK

        pltpu.sync_copy(user_ref.at[pl.ds(row0, NCHUNK)], idx_u)
        pltpu.sync_copy(item_ref.at[pl.ds(row0, NCHUNK)], idx_i)

        for idx, out in ((idx_u, fu_out), (idx_i, fi_out)):
            copies = []
            for c in range(NCHUNK):
                copies.append(pltpu.async_copy(
                    mega_ref.at[idx.at[c]],
                    rows_v.at[pl.ds(c * CHUNK, CHUNK)],
                    sem))
            for cp in copies:
                cp.wait()
            pltpu.sync_copy(rows_v, out.at[pl.ds(base, BPW)])

    return k(user, item, mega)


def _tc_mlp_body(fu, fi, w1t, w2t, b1, b2, wo, bo, out):
    gmf = fu[:, 0:D] * fi[:, 2 * D:3 * D]
    h1 = jnp.dot(fu[:, D:2 * D], w1t[...][:D],
                 preferred_element_type=jnp.float32)
    h1 = h1 + jnp.dot(fi[:, 3 * D:4 * D], w1t[...][D:],
                      preferred_element_type=jnp.float32)
    h1 = jnp.maximum(h1 + b1[...], 0.0)
    h2 = jnp.maximum(
        jnp.dot(h1, w2t[...], preferred_element_type=jnp.float32) + b2[...], 0.0)
    o = jnp.dot(gmf, wo[...][:D], preferred_element_type=jnp.float32)
    o = o + jnp.dot(h2, wo[...][D:], preferred_element_type=jnp.float32)
    out[...] = o + bo[0, 0]


def _tc_mlp(fu, fi, w1t, w2t, b1, b2, wo, bo):
    BLK = 2048
    grid = B // BLK
    full = lambda shape: pl.BlockSpec(shape, lambda i: (0, 0))
    return pl.pallas_call(
        _tc_mlp_body,
        grid=(grid,),
        in_specs=[
            pl.BlockSpec((BLK, 4 * D), lambda i: (i, 0)),
            pl.BlockSpec((BLK, 4 * D), lambda i: (i, 0)),
            full((2 * D, 64)),
            full((64, D)),
            full((1, 64)),
            full((1, D)),
            full((2 * D, 1)),
            full((1, 1)),
        ],
        out_specs=pl.BlockSpec((BLK, 1), lambda i: (i, 0)),
        out_shape=jax.ShapeDtypeStruct((B, 1), jnp.float32),
    )(fu, fi, w1t, w2t, b1, b2, wo, bo)


def kernel(user, item, gmf_user_w, gmf_item_w, mlp_user_w, mlp_item_w,
           W1, b1, W2, b2, Wo, bo):
    mega = _tc_repack(gmf_user_w, gmf_item_w, mlp_user_w, mlp_item_w)

    user2d = user.astype(jnp.int32).reshape(NW * NCHUNK, CHUNK)
    item2d = item.astype(jnp.int32).reshape(NW * NCHUNK, CHUNK)
    fu, fi = _sc_gather(user2d, item2d, mega)

    w1t = W1.T
    w2t = W2.T
    wo = Wo.T
    out = _tc_mlp(fu, fi, w1t, w2t,
                  b1.reshape(1, -1), b2.reshape(1, -1), wo, bo.reshape(1, 1))
    return out[:, 0]


# speedup vs baseline: 3.2061x; 1.0087x over previous
import functools

import jax
import jax.numpy as jnp
from jax import lax
from jax.experimental import pallas as pl
from jax.experimental.pallas import tpu as pltpu
from jax.experimental.pallas import tpu_sc as plsc

B = 16384
NU = 1000000
D = 32
NC = 2
NS = 16
NW = NC * NS
BPW = B // NW
CHUNK = 128
NCHUNK = BPW // CHUNK
RK = 16384


def _tc_repack_body(guT, muT, giT, miT, out):
    for t, r in enumerate((guT, muT, giT, miT)):
        at = jnp.transpose(r[...].astype(jnp.bfloat16))
        out[:, t * D:(t + 1) * D] = at.astype(jnp.float32)


def _tc_repack(gu, gi, mu, mi):
    grid = (NU + RK - 1) // RK
    tspec = pl.BlockSpec((D, RK), lambda i: (0, i))
    return pl.pallas_call(
        _tc_repack_body,
        grid=(grid,),
        in_specs=[tspec, tspec, tspec, tspec],
        out_specs=pl.BlockSpec((RK, 4 * D), lambda i: (i, 0)),
        out_shape=jax.ShapeDtypeStruct((NU, 4 * D), jnp.float32),
    )(gu.T, mu.T, gi.T, mi.T)


def _sc_gather(user, item, mega):
    mesh = plsc.VectorSubcoreMesh(core_axis_name="c", subcore_axis_name="s")

    @functools.partial(
        pl.kernel,
        mesh=mesh,
        compiler_params=pltpu.CompilerParams(use_tc_tiling_on_sc=True),
        out_type=(
            jax.ShapeDtypeStruct((B, 4 * D), jnp.float32),
            jax.ShapeDtypeStruct((B, 4 * D), jnp.float32),
        ),
        scratch_types=[
            pltpu.VMEM((NCHUNK, CHUNK), jnp.int32),
            pltpu.VMEM((NCHUNK, CHUNK), jnp.int32),
            pltpu.VMEM((BPW, 4 * D), jnp.float32),
            pltpu.SemaphoreType.DMA,
        ],
    )
    def k(user_ref, item_ref, mega_ref, fu_out, fi_out, idx_u, idx_i, rows_v, sem):
        wid = lax.axis_index("s") * NC + lax.axis_index("c")
        base = wid * BPW
        row0 = wid * NCHUNK

        pltpu.sync_copy(user_ref.at[pl.ds(row0, NCHUNK)], idx_u)
        pltpu.sync_copy(item_ref.at[pl.ds(row0, NCHUNK)], idx_i)

        for idx, out in ((idx_u, fu_out), (idx_i, fi_out)):
            copies = []
            for c in range(NCHUNK):
                copies.append(pltpu.async_copy(
                    mega_ref.at[idx.at[c]],
                    rows_v.at[pl.ds(c * CHUNK, CHUNK)],
                    sem))
            for cp in copies:
                cp.wait()
            pltpu.sync_copy(rows_v, out.at[pl.ds(base, BPW)])

    return k(user, item, mega)


def _tc_mlp_body(fu, fi, w1t, w2t, b1, b2, wo, bo, out):
    gmf = fu[:, 0:D] * fi[:, 2 * D:3 * D]
    h1 = jnp.dot(fu[:, D:2 * D], w1t[...][:D],
                 preferred_element_type=jnp.float32)
    h1 = h1 + jnp.dot(fi[:, 3 * D:4 * D], w1t[...][D:],
                      preferred_element_type=jnp.float32)
    h1 = jnp.maximum(h1 + b1[...], 0.0)
    h2 = jnp.maximum(
        jnp.dot(h1, w2t[...], preferred_element_type=jnp.float32) + b2[...], 0.0)
    o = jnp.dot(gmf, wo[...][:D], preferred_element_type=jnp.float32)
    o = o + jnp.dot(h2, wo[...][D:], preferred_element_type=jnp.float32)
    out[...] = o + bo[0, 0]


def _tc_mlp(fu, fi, w1t, w2t, b1, b2, wo, bo):
    BLK = 2048
    grid = B // BLK
    full = lambda shape: pl.BlockSpec(shape, lambda i: (0, 0))
    return pl.pallas_call(
        _tc_mlp_body,
        grid=(grid,),
        in_specs=[
            pl.BlockSpec((BLK, 4 * D), lambda i: (i, 0)),
            pl.BlockSpec((BLK, 4 * D), lambda i: (i, 0)),
            full((2 * D, 64)),
            full((64, D)),
            full((1, 64)),
            full((1, D)),
            full((2 * D, 1)),
            full((1, 1)),
        ],
        out_specs=pl.BlockSpec((BLK, 1), lambda i: (i, 0)),
        out_shape=jax.ShapeDtypeStruct((B, 1), jnp.float32),
    )(fu, fi, w1t, w2t, b1, b2, wo, bo)


def kernel(user, item, gmf_user_w, gmf_item_w, mlp_user_w, mlp_item_w,
           W1, b1, W2, b2, Wo, bo):
    mega = _tc_repack(gmf_user_w, gmf_item_w, mlp_user_w, mlp_item_w)

    user2d = user.astype(jnp.int32).reshape(NW * NCHUNK, CHUNK)
    item2d = item.astype(jnp.int32).reshape(NW * NCHUNK, CHUNK)
    fu, fi = _sc_gather(user2d, item2d, mega)

    w1t = W1.T
    w2t = W2.T
    wo = Wo.T
    out = _tc_mlp(fu, fi, w1t, w2t,
                  b1.reshape(1, -1), b2.reshape(1, -1), wo, bo.reshape(1, 1))
    return out[:, 0]
